# baseline (speedup 1.0000x reference)
import functools

import jax
import jax.numpy as jnp
from jax import lax
from jax.experimental import pallas as pl
from jax.experimental.pallas import tpu as pltpu
from jax.experimental.pallas import tpu_sc as plsc

K_NN = 16
BLK = 64


def _a1_body(ft_ref, w_ref, s_ref, ss_ref):
    i = pl.program_id(0)
    h = jnp.dot(ft_ref[...], w_ref[...], preferred_element_type=jnp.float32)
    cs = jnp.sum(h, axis=0)
    css = jnp.sum(h * h, axis=0)

    @pl.when(i == 0)
    def _():
        s_ref[...] = jnp.zeros_like(s_ref)
        ss_ref[...] = jnp.zeros_like(ss_ref)

    s_ref[...] += jnp.broadcast_to(cs[None, :], s_ref.shape)
    ss_ref[...] += jnp.broadcast_to(css[None, :], ss_ref.shape)


def _pre_stats(ft, w1t, n_rows, tile):
    c1 = w1t.shape[1]
    s, ss = pl.pallas_call(
        _a1_body,
        grid=(n_rows // tile,),
        in_specs=[
            pl.BlockSpec((tile, ft.shape[1]), lambda i: (i, 0)),
            pl.BlockSpec(w1t.shape, lambda i: (0, 0)),
        ],
        out_specs=[
            pl.BlockSpec((8, c1), lambda i: (0, 0)),
            pl.BlockSpec((8, c1), lambda i: (0, 0)),
        ],
        out_shape=[
            jax.ShapeDtypeStruct((8, c1), jnp.float32),
            jax.ShapeDtypeStruct((8, c1), jnp.float32),
        ],
    )(ft, w1t)
    return s[0], ss[0]


def _a2_body(ft_ref, w1_ref, b1_ref, w2_ref, b2_ref, out_ref):
    h = jnp.dot(ft_ref[...], w1_ref[...], preferred_element_type=jnp.float32)
    h = jax.nn.relu(h + b1_ref[...])
    f = jnp.dot(h, w2_ref[...], preferred_element_type=jnp.float32)
    out_ref[...] = f + b2_ref[...]


def _pre_apply(ft, w1et, b1e, w2t, b2, n_rows, tile):
    c2 = w2t.shape[1]
    return pl.pallas_call(
        _a2_body,
        grid=(n_rows // tile,),
        in_specs=[
            pl.BlockSpec((tile, ft.shape[1]), lambda i: (i, 0)),
            pl.BlockSpec(w1et.shape, lambda i: (0, 0)),
            pl.BlockSpec((1, b1e.shape[1]), lambda i: (0, 0)),
            pl.BlockSpec(w2t.shape, lambda i: (0, 0)),
            pl.BlockSpec((1, b2.shape[1]), lambda i: (0, 0)),
        ],
        out_specs=pl.BlockSpec((tile, c2), lambda i: (i, 0)),
        out_shape=jax.ShapeDtypeStruct((n_rows, c2), jnp.float32),
    )(ft, w1et, b1e, w2t, b2)


def _dist_body(q_ref, p_ref, d_ref, bm_ref, *, n):
    q = q_ref[...]
    p = p_ref[0]
    qn = jnp.sum(q * q, axis=1, keepdims=True)
    pn = jnp.sum(p * p, axis=0, keepdims=True)
    d = qn + pn - 2.0 * jnp.dot(q, p, preferred_element_type=jnp.float32)
    d_ref[...] = d
    bm_ref[...] = jnp.min(d.reshape(d.shape[0], n // BLK, BLK), axis=-1)


def _distances(q, xyzt, b, m, n, tile):
    steps_per_b = m // tile
    return pl.pallas_call(
        functools.partial(_dist_body, n=n),
        grid=(b * steps_per_b,),
        in_specs=[
            pl.BlockSpec((tile, 3), lambda i: (i, 0)),
            pl.BlockSpec((1, 3, n), lambda i, s=steps_per_b: (i // s, 0, 0)),
        ],
        out_specs=[
            pl.BlockSpec((tile, n), lambda i: (i, 0)),
            pl.BlockSpec((tile, n // BLK), lambda i: (i, 0)),
        ],
        out_shape=[
            jax.ShapeDtypeStruct((b * m, n), jnp.float32),
            jax.ShapeDtypeStruct((b * m, n // BLK), jnp.float32),
        ],
    )(q, xyzt)


def _sc_topk_gather(bm, drows, fprime, xyzt, b, m, n):
    rows = b * m
    nblk = n // BLK
    nw = 32
    rpw = rows // nw
    mesh = plsc.VectorSubcoreMesh(
        core_axis_name="c", subcore_axis_name="s", num_cores=2,
        num_subcores=16)

    def body(bm_hbm, drows_hbm, fp_hbm, xyzt_hbm, g_hbm, p_hbm, bm_v, xyz_v,
             blk_v, feat_v, pb_v, bidx_v, fidx_v, sem):
        wid = lax.axis_index("s") * 2 + lax.axis_index("c")
        base = wid * rpw
        bw = base // m
        pltpu.sync_copy(bm_hbm.at[pl.ds(base, rpw)], bm_v)
        pltpu.sync_copy(xyzt_hbm.at[bw], xyz_v)
        iota = lax.iota(jnp.int32, 16)
        zeros16 = jnp.zeros((16,), jnp.float32)
        for row in range(16):
            pb_v[row, :] = zeros16
        inf16 = jnp.full((16,), jnp.inf, jnp.float32)

        def merge(ad, ai, kd, ki):
            sd, si = plsc.sort_key_val(kd, ki)
            rd = lax.rev(sd, (0,))
            ri = lax.rev(si, (0,))
            take = ad <= rd
            md = jnp.where(take, ad, rd)
            mi = jnp.where(take, ai, ri)
            fd, fi = plsc.sort_key_val(md, mi)
            return fd, fi

        def merge_t(ad, ai, kd, ki):
            return lax.cond(jnp.any(kd < ad[15]),
                            lambda: merge(ad, ai, kd, ki),
                            lambda: (ad, ai))

        def row_step(i, carry):
            r = base + i
            ad, ai = plsc.sort_key_val(bm_v[i, pl.ds(0, 16)], iota)
            for c in range(1, nblk // 16):
                kd = bm_v[i, pl.ds(c * 16, 16)]
                ad, ai = merge_t(ad, ai, kd, iota + (c * 16))
            bidx_v[...] = lax.shift_right_logical(ai, 1) + r * (n // 128)
            pltpu.async_copy(drows_hbm.at[bidx_v], blk_v, sem).wait()
            ed = inf16
            ei = jnp.zeros((16,), jnp.int32)
            nbase_all = ai * BLK
            off_all = (ai & 1) * BLK
            for j in range(16):
                nbase = nbase_all[j]
                off = off_all[j]
                for c4 in range(BLK // 16):
                    kd = blk_v[j, pl.ds(off + c4 * 16, 16)]
                    ed, ei = merge_t(ed, ei, kd, iota + (nbase + c4 * 16))
            fidx_v[...] = ei + bw * n
            pltpu.async_copy(fp_hbm.at[fidx_v], feat_v, sem).wait()
            pltpu.sync_copy(feat_v, g_hbm.at[pl.ds(r * K_NN, K_NN)])
            for c in range(3):
                pc = plsc.load_gather(xyz_v, [jnp.full((16,), c, jnp.int32),
                                              ei])
                plsc.store_scatter(pb_v, [iota,
                                          jnp.full((16,), c, jnp.int32)], pc)
            pltpu.sync_copy(pb_v, p_hbm.at[pl.ds(r * K_NN, K_NN)])
            return carry

        lax.fori_loop(0, rpw, row_step, 0)

    fn = pl.kernel(
        body,
        out_type=(
            jax.ShapeDtypeStruct((rows * K_NN, 128), jnp.float32),
            jax.ShapeDtypeStruct((rows * K_NN, 16), jnp.float32),
        ),
        mesh=mesh,
        compiler_params=pltpu.CompilerParams(needs_layout_passes=False),
        scratch_types=[
            pltpu.VMEM((rpw, nblk), jnp.float32),
            pltpu.VMEM((3, n), jnp.float32),
            pltpu.VMEM((16, 128), jnp.float32),
            pltpu.VMEM((16, 128), jnp.float32),
            pltpu.VMEM((16, 16), jnp.float32),
            pltpu.VMEM((16,), jnp.int32),
            pltpu.VMEM((16,), jnp.int32),
            pltpu.SemaphoreType.DMA,
        ],
    )
    return fn(bm, drows, fprime, xyzt)


def _mlp1_body(g_ref, p_ref, q_ref, w1_ref, wp_ref, w1q_ref, y_ref, s_ref,
               ss_ref):
    i = pl.program_id(0)
    y = jnp.dot(g_ref[...], w1_ref[...], preferred_element_type=jnp.float32)
    y = y + jnp.dot(p_ref[...], wp_ref[...], preferred_element_type=jnp.float32)
    yq = jnp.dot(q_ref[...], w1q_ref[...], preferred_element_type=jnp.float32)
    qt = q_ref.shape[0]
    d1 = w1_ref.shape[1]
    y = (y.reshape(qt, K_NN, d1) + yq[:, None, :]).reshape(qt * K_NN, d1)
    y_ref[...] = y
    cs = jnp.sum(y, axis=0)
    css = jnp.sum(y * y, axis=0)

    @pl.when(i == 0)
    def _():
        s_ref[...] = jnp.zeros_like(s_ref)
        ss_ref[...] = jnp.zeros_like(ss_ref)

    s_ref[...] += jnp.broadcast_to(cs[None, :], s_ref.shape)
    ss_ref[...] += jnp.broadcast_to(css[None, :], ss_ref.shape)


def _mlp1(g2d, p2d, q, w1pt, wp, w1q, rows, qtile):
    d1 = w1pt.shape[1]
    return pl.pallas_call(
        _mlp1_body,
        grid=(rows // (qtile * K_NN),),
        in_specs=[
            pl.BlockSpec((qtile * K_NN, g2d.shape[1]), lambda i: (i, 0)),
            pl.BlockSpec((qtile * K_NN, p2d.shape[1]), lambda i: (i, 0)),
            pl.BlockSpec((qtile, 3), lambda i: (i, 0)),
            pl.BlockSpec(w1pt.shape, lambda i: (0, 0)),
            pl.BlockSpec(wp.shape, lambda i: (0, 0)),
            pl.BlockSpec(w1q.shape, lambda i: (0, 0)),
        ],
        out_specs=[
            pl.BlockSpec((qtile * K_NN, d1), lambda i: (i, 0)),
            pl.BlockSpec((8, d1), lambda i: (0, 0)),
            pl.BlockSpec((8, d1), lambda i: (0, 0)),
        ],
        out_shape=[
            jax.ShapeDtypeStruct((rows, d1), jnp.float32),
            jax.ShapeDtypeStruct((8, d1), jnp.float32),
            jax.ShapeDtypeStruct((8, d1), jnp.float32),
        ],
    )(g2d, p2d, q, w1pt, wp, w1q)


def _mlp2_body(y1_ref, s1_ref, t1_ref, w2_ref, ymax_ref, ymin_ref, s_ref,
               ss_ref):
    i = pl.program_id(0)
    z = jax.nn.relu(y1_ref[...] * s1_ref[...] + t1_ref[...])
    y = jnp.dot(z, w2_ref[...], preferred_element_type=jnp.float32)
    cs = jnp.sum(y, axis=0)
    css = jnp.sum(y * y, axis=0)
    qt = ymax_ref.shape[0]
    d2 = w2_ref.shape[1]
    y3 = y.reshape(qt, K_NN, d2)
    ymax_ref[...] = jnp.max(y3, axis=1)
    ymin_ref[...] = jnp.min(y3, axis=1)

    @pl.when(i == 0)
    def _():
        s_ref[...] = jnp.zeros_like(s_ref)
        ss_ref[...] = jnp.zeros_like(ss_ref)

    s_ref[...] += jnp.broadcast_to(cs[None, :], s_ref.shape)
    ss_ref[...] += jnp.broadcast_to(css[None, :], ss_ref.shape)


def _mlp2(y1, s1, t1, w2t, rows, qtile):
    d2 = w2t.shape[1]
    nq = rows // K_NN
    return pl.pallas_call(
        _mlp2_body,
        grid=(rows // (qtile * K_NN),),
        in_specs=[
            pl.BlockSpec((qtile * K_NN, y1.shape[1]), lambda i: (i, 0)),
            pl.BlockSpec((1, y1.shape[1]), lambda i: (0, 0)),
            pl.BlockSpec((1, y1.shape[1]), lambda i: (0, 0)),
            pl.BlockSpec(w2t.shape, lambda i: (0, 0)),
        ],
        out_specs=[
            pl.BlockSpec((qtile, d2), lambda i: (i, 0)),
            pl.BlockSpec((qtile, d2), lambda i: (i, 0)),
            pl.BlockSpec((8, d2), lambda i: (0, 0)),
            pl.BlockSpec((8, d2), lambda i: (0, 0)),
        ],
        out_shape=[
            jax.ShapeDtypeStruct((nq, d2), jnp.float32),
            jax.ShapeDtypeStruct((nq, d2), jnp.float32),
            jax.ShapeDtypeStruct((8, d2), jnp.float32),
            jax.ShapeDtypeStruct((8, d2), jnp.float32),
        ],
    )(y1, s1, t1, w2t)


def _fin_body(ymax_ref, ymin_ref, s2_ref, t2_ref, out_ref):
    s2 = s2_ref[...]
    t2 = t2_ref[...]
    o = jnp.where(s2 >= 0.0, ymax_ref[...] * s2 + t2, ymin_ref[...] * s2 + t2)
    o = jax.nn.relu(o)
    out_ref[...] = o.T[None]


def _finish(ymax, ymin, s2, t2, b, m, d2):
    return pl.pallas_call(
        _fin_body,
        grid=(b,),
        in_specs=[
            pl.BlockSpec((m, d2), lambda i: (i, 0)),
            pl.BlockSpec((m, d2), lambda i: (i, 0)),
            pl.BlockSpec((1, d2), lambda i: (0, 0)),
            pl.BlockSpec((1, d2), lambda i: (0, 0)),
        ],
        out_specs=pl.BlockSpec((1, d2, m), lambda i: (i, 0, 0)),
        out_shape=jax.ShapeDtypeStruct((b, d2, m), jnp.float32),
    )(ymax, ymin, s2, t2)


def _bn_fold(s, ss, count, g, bt):
    mean = s / count
    var = ss / count - mean * mean
    scale = g * lax.rsqrt(var + 1e-5)
    shift = bt - mean * scale
    return scale, shift


def kernel(new_xyz, xyz, feat, pre_W1, pre_b1, pre_g1, pre_bt1, pre_W2,
           pre_b2, W1, g1, bt1, W2, g2, bt2):
    B, M, _ = new_xyz.shape
    N = xyz.shape[1]
    Cin = feat.shape[1]
    D1 = W1.shape[0]
    D2 = W2.shape[0]

    ft = feat.transpose(0, 2, 1).reshape(B * N, Cin)
    xyzt = xyz.transpose(0, 2, 1)
    q = new_xyz.reshape(B * M, 3)

    s, ss = _pre_stats(ft, pre_W1.T, B * N, 1024)
    sc1, sh1 = _bn_fold(s, ss, float(B * N), pre_g1, pre_bt1)
    w1et = (pre_W1 * sc1[:, None]).T
    fprime = _pre_apply(ft, w1et, sh1[None, :], pre_W2.T,
                        pre_b2[None, :], B * N, 1024)

    d, bm = _distances(q, xyzt, B, M, N, 256)
    drows = d.reshape(B * M * (N // 128), 128)

    g_rows, p_rows = _sc_topk_gather(bm, drows, fprime, xyzt, B, M, N)

    w1pt = W1[:, 6:134].T
    wp = jnp.zeros((16, D1), jnp.float32)
    wp = wp.at[0:3, :].set(W1[:, 0:3].T)
    w1q = -W1[:, 0:3].T + W1[:, 3:6].T

    y1, s1sum, s1ss = _mlp1(g_rows, p_rows, q, w1pt, wp, w1q,
                            B * M * K_NN, 128)
    cnt = float(B * M * K_NN)
    sc_1, sh_1 = _bn_fold(s1sum[0], s1ss[0], cnt, g1, bt1)
    ymax, ymin, s2sum, s2ss = _mlp2(y1, sc_1[None, :], sh_1[None, :], W2.T,
                                    B * M * K_NN, 128)
    sc_2, sh_2 = _bn_fold(s2sum[0], s2ss[0], cnt, g2, bt2)
    return _finish(ymax, ymin, sc_2[None, :], sh_2[None, :], B, M, D2)


# speedup vs baseline: 1.7475x; 1.7475x over previous
import functools

import jax
import jax.numpy as jnp
from jax import lax
from jax.experimental import pallas as pl
from jax.experimental.pallas import tpu as pltpu
from jax.experimental.pallas import tpu_sc as plsc

K_NN = 16
BLK = 64


def _a1_body(ft_ref, w_ref, s_ref, ss_ref):
    i = pl.program_id(0)
    h = jnp.dot(ft_ref[...], w_ref[...], preferred_element_type=jnp.float32)
    cs = jnp.sum(h, axis=0)
    css = jnp.sum(h * h, axis=0)

    @pl.when(i == 0)
    def _():
        s_ref[...] = jnp.zeros_like(s_ref)
        ss_ref[...] = jnp.zeros_like(ss_ref)

    s_ref[...] += jnp.broadcast_to(cs[None, :], s_ref.shape)
    ss_ref[...] += jnp.broadcast_to(css[None, :], ss_ref.shape)


def _pre_stats(ft, w1t, n_rows, tile):
    c1 = w1t.shape[1]
    s, ss = pl.pallas_call(
        _a1_body,
        grid=(n_rows // tile,),
        in_specs=[
            pl.BlockSpec((tile, ft.shape[1]), lambda i: (i, 0)),
            pl.BlockSpec(w1t.shape, lambda i: (0, 0)),
        ],
        out_specs=[
            pl.BlockSpec((8, c1), lambda i: (0, 0)),
            pl.BlockSpec((8, c1), lambda i: (0, 0)),
        ],
        out_shape=[
            jax.ShapeDtypeStruct((8, c1), jnp.float32),
            jax.ShapeDtypeStruct((8, c1), jnp.float32),
        ],
    )(ft, w1t)
    return s[0], ss[0]


def _a2_body(ft_ref, w1_ref, b1_ref, w2_ref, b2_ref, out_ref):
    h = jnp.dot(ft_ref[...], w1_ref[...], preferred_element_type=jnp.float32)
    h = jax.nn.relu(h + b1_ref[...])
    f = jnp.dot(h, w2_ref[...], preferred_element_type=jnp.float32)
    out_ref[...] = f + b2_ref[...]


def _pre_apply(ft, w1et, b1e, w2t, b2, n_rows, tile):
    c2 = w2t.shape[1]
    return pl.pallas_call(
        _a2_body,
        grid=(n_rows // tile,),
        in_specs=[
            pl.BlockSpec((tile, ft.shape[1]), lambda i: (i, 0)),
            pl.BlockSpec(w1et.shape, lambda i: (0, 0)),
            pl.BlockSpec((1, b1e.shape[1]), lambda i: (0, 0)),
            pl.BlockSpec(w2t.shape, lambda i: (0, 0)),
            pl.BlockSpec((1, b2.shape[1]), lambda i: (0, 0)),
        ],
        out_specs=pl.BlockSpec((tile, c2), lambda i: (i, 0)),
        out_shape=jax.ShapeDtypeStruct((n_rows, c2), jnp.float32),
    )(ft, w1et, b1e, w2t, b2)


def _dist_body(q_ref, p_ref, d_ref, bm_ref, *, n):
    q = q_ref[...]
    p = p_ref[0]
    qn = jnp.sum(q * q, axis=1, keepdims=True)
    pn = jnp.sum(p * p, axis=0, keepdims=True)
    d = qn + pn - 2.0 * jnp.dot(q, p, preferred_element_type=jnp.float32)
    d_ref[...] = d
    bm_ref[...] = jnp.min(d.reshape(d.shape[0], n // BLK, BLK), axis=-1)


def _distances(q, xyzt, b, m, n, tile):
    steps_per_b = m // tile
    return pl.pallas_call(
        functools.partial(_dist_body, n=n),
        grid=(b * steps_per_b,),
        in_specs=[
            pl.BlockSpec((tile, 3), lambda i: (i, 0)),
            pl.BlockSpec((1, 3, n), lambda i, s=steps_per_b: (i // s, 0, 0)),
        ],
        out_specs=[
            pl.BlockSpec((tile, n), lambda i: (i, 0)),
            pl.BlockSpec((tile, n // BLK), lambda i: (i, 0)),
        ],
        out_shape=[
            jax.ShapeDtypeStruct((b * m, n), jnp.float32),
            jax.ShapeDtypeStruct((b * m, n // BLK), jnp.float32),
        ],
    )(q, xyzt)


def _sc_topk_gather(bm, drows, fprime, xyzt, b, m, n):
    rows = b * m
    nblk = n // BLK
    nw = 32
    rpw = rows // nw
    mesh = plsc.VectorSubcoreMesh(
        core_axis_name="c", subcore_axis_name="s", num_cores=2,
        num_subcores=16)

    grp = 4
    ngrp = rpw // grp

    def body(bm_hbm, drows_hbm, fp_hbm, xyzt_hbm, g_hbm, p_hbm, xyz_v,
             bmg0, bmg1, blk0, blk1, gbig0, gbig1, pbig0, pbig1,
             aia0, aia1, bidx0, bidx1, fidx0, fidx1,
             semb0, semb1, semf0, semf1):
        wid = lax.axis_index("s") * 2 + lax.axis_index("c")
        base = wid * rpw
        bw = base // m
        pltpu.sync_copy(xyzt_hbm.at[bw], xyz_v)
        iota = lax.iota(jnp.int32, 16)
        zeros16 = jnp.zeros((16,), jnp.float32)
        for row in range(grp * 16):
            pbig0[row, :] = zeros16
            pbig1[row, :] = zeros16
        inf16 = jnp.full((16,), jnp.inf, jnp.float32)

        def merge(ad, ai, kd, ki):
            sd, si = plsc.sort_key_val(kd, ki)
            rd = lax.rev(sd, (0,))
            ri = lax.rev(si, (0,))
            take = ad <= rd
            md = jnp.where(take, ad, rd)
            mi = jnp.where(take, ai, ri)
            fd, fi = plsc.sort_key_val(md, mi)
            return fd, fi

        def phase_a(g, bmg, aia, bidx, blk, semb):
            rowb = base + g * grp
            pltpu.sync_copy(bm_hbm.at[pl.ds(rowb, grp)], bmg)

            def arow(j, carry):
                ad, ai = plsc.sort_key_val(bmg[j, pl.ds(0, 16)], iota)
                for c in range(1, nblk // 16):
                    kd = bmg[j, pl.ds(c * 16, 16)]
                    ad, ai = merge(ad, ai, kd, iota + (c * 16))
                aia[pl.ds(j * 16, 16)] = ai
                bidx[pl.ds(j * 16, 16)] = (
                    lax.shift_right_logical(ai, 1) + (rowb + j) * (n // 128))
                return carry

            lax.fori_loop(0, grp, arow, 0)
            pltpu.async_copy(drows_hbm.at[bidx], blk, semb)

        def phase_b(g, aia, blk, pbig, fidx, gbig, semf):

            def brow(j, carry):
                ai = aia[pl.ds(j * 16, 16)]
                ed = inf16
                ei = jnp.zeros((16,), jnp.int32)
                nbase_all = ai * BLK
                off_all = (ai & 1) * BLK
                for jj in range(16):
                    nbase = nbase_all[jj]
                    off = off_all[jj]
                    for c4 in range(BLK // 16):
                        kd = blk[j * 16 + jj, pl.ds(off + c4 * 16, 16)]
                        ed, ei = merge(ed, ei, kd, iota + (nbase + c4 * 16))
                fidx[pl.ds(j * 16, 16)] = ei + bw * n
                for c in range(3):
                    pc = plsc.load_gather(
                        xyz_v, [jnp.full((16,), c, jnp.int32), ei])
                    plsc.store_scatter(
                        pbig, [iota + j * 16,
                               jnp.full((16,), c, jnp.int32)], pc)
                return carry

            lax.fori_loop(0, grp, brow, 0)
            return pltpu.async_copy(fp_hbm.at[fidx], gbig, semf)

        def out_copies(g, gbig, pbig):
            rowb = base + g * grp
            pltpu.sync_copy(gbig, g_hbm.at[pl.ds(rowb * K_NN, grp * K_NN)])
            pltpu.sync_copy(pbig, p_hbm.at[pl.ds(rowb * K_NN, grp * K_NN)])

        def drain_b(blk, semb):
            pltpu.make_async_copy(
                drows_hbm.at[pl.ds(0, grp * 16)], blk, semb).wait()

        phase_a(0, bmg0, aia0, bidx0, blk0, semb0)

        def step(t, carry):
            a = 2 * t
            b = 2 * t + 1
            nxt = jnp.minimum(a + 2, ngrp - 1)
            phase_a(b, bmg1, aia1, bidx1, blk1, semb1)
            drain_b(blk0, semb0)
            ha = phase_b(a, aia0, blk0, pbig0, fidx0, gbig0, semf0)
            phase_a(nxt, bmg0, aia0, bidx0, blk0, semb0)
            drain_b(blk1, semb1)
            hb = phase_b(b, aia1, blk1, pbig1, fidx1, gbig1, semf1)
            ha.wait()
            out_copies(a, gbig0, pbig0)
            hb.wait()
            out_copies(b, gbig1, pbig1)
            return carry

        lax.fori_loop(0, ngrp // 2, step, 0)
        drain_b(blk0, semb0)

    fn = pl.kernel(
        body,
        out_type=(
            jax.ShapeDtypeStruct((rows * K_NN, 128), jnp.float32),
            jax.ShapeDtypeStruct((rows * K_NN, 16), jnp.float32),
        ),
        mesh=mesh,
        compiler_params=pltpu.CompilerParams(needs_layout_passes=False),
        scratch_types=[
            pltpu.VMEM((3, n), jnp.float32),
            pltpu.VMEM((grp, nblk), jnp.float32),
            pltpu.VMEM((grp, nblk), jnp.float32),
            pltpu.VMEM((grp * 16, 128), jnp.float32),
            pltpu.VMEM((grp * 16, 128), jnp.float32),
            pltpu.VMEM((grp * 16, 128), jnp.float32),
            pltpu.VMEM((grp * 16, 128), jnp.float32),
            pltpu.VMEM((grp * 16, 16), jnp.float32),
            pltpu.VMEM((grp * 16, 16), jnp.float32),
            pltpu.VMEM((grp * 16,), jnp.int32),
            pltpu.VMEM((grp * 16,), jnp.int32),
            pltpu.VMEM((grp * 16,), jnp.int32),
            pltpu.VMEM((grp * 16,), jnp.int32),
            pltpu.VMEM((grp * 16,), jnp.int32),
            pltpu.VMEM((grp * 16,), jnp.int32),
            pltpu.SemaphoreType.DMA,
            pltpu.SemaphoreType.DMA,
            pltpu.SemaphoreType.DMA,
            pltpu.SemaphoreType.DMA,
        ],
    )
    return fn(bm, drows, fprime, xyzt)


def _mlp1_body(g_ref, p_ref, q_ref, w1_ref, wp_ref, w1q_ref, y_ref, s_ref,
               ss_ref):
    i = pl.program_id(0)
    y = jnp.dot(g_ref[...], w1_ref[...], preferred_element_type=jnp.float32)
    y = y + jnp.dot(p_ref[...], wp_ref[...], preferred_element_type=jnp.float32)
    yq = jnp.dot(q_ref[...], w1q_ref[...], preferred_element_type=jnp.float32)
    qt = q_ref.shape[0]
    d1 = w1_ref.shape[1]
    y = (y.reshape(qt, K_NN, d1) + yq[:, None, :]).reshape(qt * K_NN, d1)
    y_ref[...] = y
    cs = jnp.sum(y, axis=0)
    css = jnp.sum(y * y, axis=0)

    @pl.when(i == 0)
    def _():
        s_ref[...] = jnp.zeros_like(s_ref)
        ss_ref[...] = jnp.zeros_like(ss_ref)

    s_ref[...] += jnp.broadcast_to(cs[None, :], s_ref.shape)
    ss_ref[...] += jnp.broadcast_to(css[None, :], ss_ref.shape)


def _mlp1(g2d, p2d, q, w1pt, wp, w1q, rows, qtile):
    d1 = w1pt.shape[1]
    return pl.pallas_call(
        _mlp1_body,
        grid=(rows // (qtile * K_NN),),
        in_specs=[
            pl.BlockSpec((qtile * K_NN, g2d.shape[1]), lambda i: (i, 0)),
            pl.BlockSpec((qtile * K_NN, p2d.shape[1]), lambda i: (i, 0)),
            pl.BlockSpec((qtile, 3), lambda i: (i, 0)),
            pl.BlockSpec(w1pt.shape, lambda i: (0, 0)),
            pl.BlockSpec(wp.shape, lambda i: (0, 0)),
            pl.BlockSpec(w1q.shape, lambda i: (0, 0)),
        ],
        out_specs=[
            pl.BlockSpec((qtile * K_NN, d1), lambda i: (i, 0)),
            pl.BlockSpec((8, d1), lambda i: (0, 0)),
            pl.BlockSpec((8, d1), lambda i: (0, 0)),
        ],
        out_shape=[
            jax.ShapeDtypeStruct((rows, d1), jnp.float32),
            jax.ShapeDtypeStruct((8, d1), jnp.float32),
            jax.ShapeDtypeStruct((8, d1), jnp.float32),
        ],
    )(g2d, p2d, q, w1pt, wp, w1q)


def _mlp2_body(y1_ref, s1_ref, t1_ref, w2_ref, ymax_ref, ymin_ref, s_ref,
               ss_ref):
    i = pl.program_id(0)
    z = jax.nn.relu(y1_ref[...] * s1_ref[...] + t1_ref[...])
    y = jnp.dot(z, w2_ref[...], preferred_element_type=jnp.float32)
    cs = jnp.sum(y, axis=0)
    css = jnp.sum(y * y, axis=0)
    qt = ymax_ref.shape[0]
    d2 = w2_ref.shape[1]
    y3 = y.reshape(qt, K_NN, d2)
    ymax_ref[...] = jnp.max(y3, axis=1)
    ymin_ref[...] = jnp.min(y3, axis=1)

    @pl.when(i == 0)
    def _():
        s_ref[...] = jnp.zeros_like(s_ref)
        ss_ref[...] = jnp.zeros_like(ss_ref)

    s_ref[...] += jnp.broadcast_to(cs[None, :], s_ref.shape)
    ss_ref[...] += jnp.broadcast_to(css[None, :], ss_ref.shape)


def _mlp2(y1, s1, t1, w2t, rows, qtile):
    d2 = w2t.shape[1]
    nq = rows // K_NN
    return pl.pallas_call(
        _mlp2_body,
        grid=(rows // (qtile * K_NN),),
        in_specs=[
            pl.BlockSpec((qtile * K_NN, y1.shape[1]), lambda i: (i, 0)),
            pl.BlockSpec((1, y1.shape[1]), lambda i: (0, 0)),
            pl.BlockSpec((1, y1.shape[1]), lambda i: (0, 0)),
            pl.BlockSpec(w2t.shape, lambda i: (0, 0)),
        ],
        out_specs=[
            pl.BlockSpec((qtile, d2), lambda i: (i, 0)),
            pl.BlockSpec((qtile, d2), lambda i: (i, 0)),
            pl.BlockSpec((8, d2), lambda i: (0, 0)),
            pl.BlockSpec((8, d2), lambda i: (0, 0)),
        ],
        out_shape=[
            jax.ShapeDtypeStruct((nq, d2), jnp.float32),
            jax.ShapeDtypeStruct((nq, d2), jnp.float32),
            jax.ShapeDtypeStruct((8, d2), jnp.float32),
            jax.ShapeDtypeStruct((8, d2), jnp.float32),
        ],
    )(y1, s1, t1, w2t)


def _fin_body(ymax_ref, ymin_ref, s2_ref, t2_ref, out_ref):
    s2 = s2_ref[...]
    t2 = t2_ref[...]
    o = jnp.where(s2 >= 0.0, ymax_ref[...] * s2 + t2, ymin_ref[...] * s2 + t2)
    o = jax.nn.relu(o)
    out_ref[...] = o.T[None]


def _finish(ymax, ymin, s2, t2, b, m, d2):
    return pl.pallas_call(
        _fin_body,
        grid=(b,),
        in_specs=[
            pl.BlockSpec((m, d2), lambda i: (i, 0)),
            pl.BlockSpec((m, d2), lambda i: (i, 0)),
            pl.BlockSpec((1, d2), lambda i: (0, 0)),
            pl.BlockSpec((1, d2), lambda i: (0, 0)),
        ],
        out_specs=pl.BlockSpec((1, d2, m), lambda i: (i, 0, 0)),
        out_shape=jax.ShapeDtypeStruct((b, d2, m), jnp.float32),
    )(ymax, ymin, s2, t2)


def _bn_fold(s, ss, count, g, bt):
    mean = s / count
    var = ss / count - mean * mean
    scale = g * lax.rsqrt(var + 1e-5)
    shift = bt - mean * scale
    return scale, shift


def kernel(new_xyz, xyz, feat, pre_W1, pre_b1, pre_g1, pre_bt1, pre_W2,
           pre_b2, W1, g1, bt1, W2, g2, bt2):
    B, M, _ = new_xyz.shape
    N = xyz.shape[1]
    Cin = feat.shape[1]
    D1 = W1.shape[0]
    D2 = W2.shape[0]

    ft = feat.transpose(0, 2, 1).reshape(B * N, Cin)
    xyzt = xyz.transpose(0, 2, 1)
    q = new_xyz.reshape(B * M, 3)

    s, ss = _pre_stats(ft, pre_W1.T, B * N, 1024)
    sc1, sh1 = _bn_fold(s, ss, float(B * N), pre_g1, pre_bt1)
    w1et = (pre_W1 * sc1[:, None]).T
    fprime = _pre_apply(ft, w1et, sh1[None, :], pre_W2.T,
                        pre_b2[None, :], B * N, 1024)

    d, bm = _distances(q, xyzt, B, M, N, 256)
    drows = d.reshape(B * M * (N // 128), 128)

    g_rows, p_rows = _sc_topk_gather(bm, drows, fprime, xyzt, B, M, N)

    w1pt = W1[:, 6:134].T
    wp = jnp.zeros((16, D1), jnp.float32)
    wp = wp.at[0:3, :].set(W1[:, 0:3].T)
    w1q = -W1[:, 0:3].T + W1[:, 3:6].T

    y1, s1sum, s1ss = _mlp1(g_rows, p_rows, q, w1pt, wp, w1q,
                            B * M * K_NN, 128)
    cnt = float(B * M * K_NN)
    sc_1, sh_1 = _bn_fold(s1sum[0], s1ss[0], cnt, g1, bt1)
    ymax, ymin, s2sum, s2ss = _mlp2(y1, sc_1[None, :], sh_1[None, :], W2.T,
                                    B * M * K_NN, 128)
    sc_2, sh_2 = _bn_fold(s2sum[0], s2ss[0], cnt, g2, bt2)
    return _finish(ymax, ymin, sc_2[None, :], sh_2[None, :], B, M, D2)


# speedup vs baseline: 1.9103x; 1.0931x over previous
import functools

import jax
import jax.numpy as jnp
from jax import lax
from jax.experimental import pallas as pl
from jax.experimental.pallas import tpu as pltpu
from jax.experimental.pallas import tpu_sc as plsc

K_NN = 16
BLK = 64


def _a1_body(ft_ref, w_ref, s_ref, ss_ref):
    i = pl.program_id(0)
    h = jnp.dot(ft_ref[...], w_ref[...], preferred_element_type=jnp.float32)
    cs = jnp.sum(h, axis=0)
    css = jnp.sum(h * h, axis=0)

    @pl.when(i == 0)
    def _():
        s_ref[...] = jnp.zeros_like(s_ref)
        ss_ref[...] = jnp.zeros_like(ss_ref)

    s_ref[...] += jnp.broadcast_to(cs[None, :], s_ref.shape)
    ss_ref[...] += jnp.broadcast_to(css[None, :], ss_ref.shape)


def _pre_stats(ft, w1t, n_rows, tile):
    c1 = w1t.shape[1]
    s, ss = pl.pallas_call(
        _a1_body,
        grid=(n_rows // tile,),
        in_specs=[
            pl.BlockSpec((tile, ft.shape[1]), lambda i: (i, 0)),
            pl.BlockSpec(w1t.shape, lambda i: (0, 0)),
        ],
        out_specs=[
            pl.BlockSpec((8, c1), lambda i: (0, 0)),
            pl.BlockSpec((8, c1), lambda i: (0, 0)),
        ],
        out_shape=[
            jax.ShapeDtypeStruct((8, c1), jnp.float32),
            jax.ShapeDtypeStruct((8, c1), jnp.float32),
        ],
    )(ft, w1t)
    return s, ss


def _a2_body(ft_ref, w1t_ref, s_ref, ss_ref, g_ref, bt_ref, w2_ref, b2_ref,
             out_ref, *, count):
    mean = s_ref[0:1, :] / count
    var = ss_ref[0:1, :] / count - mean * mean
    scale = g_ref[...] * lax.rsqrt(var + 1e-5)
    shift = bt_ref[...] - mean * scale
    w1 = w1t_ref[...] * scale
    h = jnp.dot(ft_ref[...], w1, preferred_element_type=jnp.float32)
    h = jax.nn.relu(h + shift)
    f = jnp.dot(h, w2_ref[...], preferred_element_type=jnp.float32)
    out_ref[...] = f + b2_ref[...]


def _pre_apply(ft, w1t, s, ss, g, bt, w2t, b2, n_rows, tile):
    c1 = w1t.shape[1]
    c2 = w2t.shape[1]
    return pl.pallas_call(
        functools.partial(_a2_body, count=float(n_rows)),
        grid=(n_rows // tile,),
        in_specs=[
            pl.BlockSpec((tile, ft.shape[1]), lambda i: (i, 0)),
            pl.BlockSpec(w1t.shape, lambda i: (0, 0)),
            pl.BlockSpec((8, c1), lambda i: (0, 0)),
            pl.BlockSpec((8, c1), lambda i: (0, 0)),
            pl.BlockSpec((1, c1), lambda i: (0, 0)),
            pl.BlockSpec((1, c1), lambda i: (0, 0)),
            pl.BlockSpec(w2t.shape, lambda i: (0, 0)),
            pl.BlockSpec((1, c2), lambda i: (0, 0)),
        ],
        out_specs=pl.BlockSpec((tile, c2), lambda i: (i, 0)),
        out_shape=jax.ShapeDtypeStruct((n_rows, c2), jnp.float32),
    )(ft, w1t, s, ss, g, bt, w2t, b2)


def _dist_body(q_ref, p_ref, d_ref):
    q = q_ref[...]
    p = p_ref[0]
    qn = jnp.sum(q * q, axis=1, keepdims=True)
    pn = jnp.sum(p * p, axis=0, keepdims=True)
    d_ref[...] = qn + pn - 2.0 * jnp.dot(
        q, p, preferred_element_type=jnp.float32)


def _distances(q, xyzt, b, m, n, tile):
    steps_per_b = m // tile
    return pl.pallas_call(
        _dist_body,
        grid=(b * steps_per_b,),
        in_specs=[
            pl.BlockSpec((tile, 3), lambda i: (i, 0)),
            pl.BlockSpec((1, 3, n), lambda i, s=steps_per_b: (i // s, 0, 0)),
        ],
        out_specs=pl.BlockSpec((tile, n), lambda i: (i, 0)),
        out_shape=jax.ShapeDtypeStruct((b * m, n), jnp.float32),
    )(q, xyzt)


def _bmt_body(p_ref, qt_ref, bm_ref, *, n):
    p = p_ref[0]
    qt = qt_ref[0]
    pn = jnp.sum(p * p, axis=1, keepdims=True)
    e = pn - 2.0 * jnp.dot(p, qt, preferred_element_type=jnp.float32)
    mt = e.shape[1]
    bm = jnp.min(e.reshape(n // BLK, BLK, mt), axis=1)
    qn = jnp.sum(qt * qt, axis=0, keepdims=True)
    bm_ref[...] = (bm + qn)[None]


def _blockmins(xyz, qt, b, m, n, mt):
    steps_per_b = m // mt
    return pl.pallas_call(
        functools.partial(_bmt_body, n=n),
        grid=(b * steps_per_b,),
        in_specs=[
            pl.BlockSpec((1, n, 3), lambda i, s=steps_per_b: (i // s, 0, 0)),
            pl.BlockSpec((1, 3, mt),
                         lambda i, s=steps_per_b: (i // s, 0, i % s)),
        ],
        out_specs=pl.BlockSpec((1, n // BLK, mt),
                               lambda i, s=steps_per_b: (i // s, 0, i % s)),
        out_shape=jax.ShapeDtypeStruct((b, n // BLK, m), jnp.float32),
    )(xyz, qt)


def _sc_topk_gather(bm, drows, fprime, xyzt, b, m, n):
    rows = b * m
    nblk = n // BLK
    nw = 32
    rpw = rows // nw
    mesh = plsc.VectorSubcoreMesh(
        core_axis_name="c", subcore_axis_name="s", num_cores=2,
        num_subcores=16)

    grp = 4
    ngrp = rpw // grp

    def body(bm_hbm, drows_hbm, fp_hbm, xyzt_hbm, g_hbm, p_hbm, xyz_v,
             bmg0, bmg1, blk0, blk1, gbig0, gbig1, pbig0, pbig1,
             aia0, aia1, bidx0, bidx1, fidx0, fidx1,
             semb0, semb1, semf0, semf1):
        wid = lax.axis_index("s") * 2 + lax.axis_index("c")
        base = wid * rpw
        bw = base // m
        pltpu.sync_copy(xyzt_hbm.at[bw], xyz_v)
        iota = lax.iota(jnp.int32, 16)
        zeros16 = jnp.zeros((16,), jnp.float32)
        for row in range(grp * 16):
            pbig0[row, :] = zeros16
            pbig1[row, :] = zeros16
        inf16 = jnp.full((16,), jnp.inf, jnp.float32)

        def merge(ad, ai, kd, ki):
            sd, si = plsc.sort_key_val(kd, ki)
            rd = lax.rev(sd, (0,))
            ri = lax.rev(si, (0,))
            take = ad <= rd
            md = jnp.where(take, ad, rd)
            mi = jnp.where(take, ai, ri)
            fd, fi = plsc.sort_key_val(md, mi)
            return fd, fi

        def phase_a(g, bmg, aia, bidx, blk, semb):
            rowb = base + g * grp
            pltpu.sync_copy(bm_hbm.at[pl.ds(rowb, grp)], bmg)

            def arow(j, carry):
                ad, ai = plsc.sort_key_val(bmg[j, pl.ds(0, 16)], iota)
                for c in range(1, nblk // 16):
                    kd = bmg[j, pl.ds(c * 16, 16)]
                    ad, ai = merge(ad, ai, kd, iota + (c * 16))
                aia[pl.ds(j * 16, 16)] = ai
                bidx[pl.ds(j * 16, 16)] = (
                    lax.shift_right_logical(ai, 1) + (rowb + j) * (n // 128))
                return carry

            lax.fori_loop(0, grp, arow, 0)
            pltpu.async_copy(drows_hbm.at[bidx], blk, semb)

        def phase_b(g, aia, blk, pbig, fidx, gbig, semf):

            def brow(j, carry):
                ai = aia[pl.ds(j * 16, 16)]
                ed = inf16
                ei = jnp.zeros((16,), jnp.int32)
                nbase_all = ai * BLK
                off_all = (ai & 1) * BLK
                for jj in range(16):
                    nbase = nbase_all[jj]
                    off = off_all[jj]
                    for c4 in range(BLK // 16):
                        kd = blk[j * 16 + jj, pl.ds(off + c4 * 16, 16)]
                        ed, ei = merge(ed, ei, kd, iota + (nbase + c4 * 16))
                fidx[pl.ds(j * 16, 16)] = ei + bw * n
                for c in range(3):
                    pc = plsc.load_gather(
                        xyz_v, [jnp.full((16,), c, jnp.int32), ei])
                    plsc.store_scatter(
                        pbig, [iota + j * 16,
                               jnp.full((16,), c, jnp.int32)], pc)
                return carry

            lax.fori_loop(0, grp, brow, 0)
            return pltpu.async_copy(fp_hbm.at[fidx], gbig, semf)

        def out_copies(g, gbig, pbig):
            rowb = base + g * grp
            pltpu.sync_copy(gbig, g_hbm.at[pl.ds(rowb * K_NN, grp * K_NN)])
            pltpu.sync_copy(pbig, p_hbm.at[pl.ds(rowb * K_NN, grp * K_NN)])

        def drain_b(blk, semb):
            pltpu.make_async_copy(
                drows_hbm.at[pl.ds(0, grp * 16)], blk, semb).wait()

        phase_a(0, bmg0, aia0, bidx0, blk0, semb0)

        def step(t, carry):
            a = 2 * t
            b = 2 * t + 1
            nxt = jnp.minimum(a + 2, ngrp - 1)
            phase_a(b, bmg1, aia1, bidx1, blk1, semb1)
            drain_b(blk0, semb0)
            ha = phase_b(a, aia0, blk0, pbig0, fidx0, gbig0, semf0)
            phase_a(nxt, bmg0, aia0, bidx0, blk0, semb0)
            drain_b(blk1, semb1)
            hb = phase_b(b, aia1, blk1, pbig1, fidx1, gbig1, semf1)
            ha.wait()
            out_copies(a, gbig0, pbig0)
            hb.wait()
            out_copies(b, gbig1, pbig1)
            return carry

        lax.fori_loop(0, ngrp // 2, step, 0)
        drain_b(blk0, semb0)

    fn = pl.kernel(
        body,
        out_type=(
            jax.ShapeDtypeStruct((rows * K_NN, 128), jnp.float32),
            jax.ShapeDtypeStruct((rows * K_NN, 16), jnp.float32),
        ),
        mesh=mesh,
        compiler_params=pltpu.CompilerParams(needs_layout_passes=False),
        scratch_types=[
            pltpu.VMEM((3, n), jnp.float32),
            pltpu.VMEM((grp, nblk), jnp.float32),
            pltpu.VMEM((grp, nblk), jnp.float32),
            pltpu.VMEM((grp * 16, 128), jnp.float32),
            pltpu.VMEM((grp * 16, 128), jnp.float32),
            pltpu.VMEM((grp * 16, 128), jnp.float32),
            pltpu.VMEM((grp * 16, 128), jnp.float32),
            pltpu.VMEM((grp * 16, 16), jnp.float32),
            pltpu.VMEM((grp * 16, 16), jnp.float32),
            pltpu.VMEM((grp * 16,), jnp.int32),
            pltpu.VMEM((grp * 16,), jnp.int32),
            pltpu.VMEM((grp * 16,), jnp.int32),
            pltpu.VMEM((grp * 16,), jnp.int32),
            pltpu.VMEM((grp * 16,), jnp.int32),
            pltpu.VMEM((grp * 16,), jnp.int32),
            pltpu.SemaphoreType.DMA,
            pltpu.SemaphoreType.DMA,
            pltpu.SemaphoreType.DMA,
            pltpu.SemaphoreType.DMA,
        ],
    )
    return fn(bm, drows, fprime, xyzt)


def _mlp1_body(g_ref, p_ref, q_ref, w1_ref, wp_ref, w1q_ref, y_ref, s_ref,
               ss_ref):
    i = pl.program_id(0)
    y = jnp.dot(g_ref[...], w1_ref[...], preferred_element_type=jnp.float32)
    y = y + jnp.dot(p_ref[...], wp_ref[...], preferred_element_type=jnp.float32)
    yq = jnp.dot(q_ref[...], w1q_ref[...], preferred_element_type=jnp.float32)
    qt = q_ref.shape[0]
    d1 = w1_ref.shape[1]
    y = (y.reshape(qt, K_NN, d1) + yq[:, None, :]).reshape(qt * K_NN, d1)
    y_ref[...] = y
    cs = jnp.sum(y, axis=0)
    css = jnp.sum(y * y, axis=0)

    @pl.when(i == 0)
    def _():
        s_ref[...] = jnp.zeros_like(s_ref)
        ss_ref[...] = jnp.zeros_like(ss_ref)

    s_ref[...] += jnp.broadcast_to(cs[None, :], s_ref.shape)
    ss_ref[...] += jnp.broadcast_to(css[None, :], ss_ref.shape)


def _mlp1(g2d, p2d, q, w1pt, wp, w1q, rows, qtile):
    d1 = w1pt.shape[1]
    return pl.pallas_call(
        _mlp1_body,
        grid=(rows // (qtile * K_NN),),
        in_specs=[
            pl.BlockSpec((qtile * K_NN, g2d.shape[1]), lambda i: (i, 0)),
            pl.BlockSpec((qtile * K_NN, p2d.shape[1]), lambda i: (i, 0)),
            pl.BlockSpec((qtile, 3), lambda i: (i, 0)),
            pl.BlockSpec(w1pt.shape, lambda i: (0, 0)),
            pl.BlockSpec(wp.shape, lambda i: (0, 0)),
            pl.BlockSpec(w1q.shape, lambda i: (0, 0)),
        ],
        out_specs=[
            pl.BlockSpec((qtile * K_NN, d1), lambda i: (i, 0)),
            pl.BlockSpec((8, d1), lambda i: (0, 0)),
            pl.BlockSpec((8, d1), lambda i: (0, 0)),
        ],
        out_shape=[
            jax.ShapeDtypeStruct((rows, d1), jnp.float32),
            jax.ShapeDtypeStruct((8, d1), jnp.float32),
            jax.ShapeDtypeStruct((8, d1), jnp.float32),
        ],
    )(g2d, p2d, q, w1pt, wp, w1q)


def _mlp2_body(y1_ref, ps_ref, pss_ref, g_ref, bt_ref, w2_ref, ymax_ref,
               ymin_ref, s_ref, ss_ref, *, count):
    i = pl.program_id(0)
    mean = ps_ref[0:1, :] / count
    var = pss_ref[0:1, :] / count - mean * mean
    scale = g_ref[...] * lax.rsqrt(var + 1e-5)
    shift = bt_ref[...] - mean * scale
    z = jax.nn.relu(y1_ref[...] * scale + shift)
    y = jnp.dot(z, w2_ref[...], preferred_element_type=jnp.float32)
    cs = jnp.sum(y, axis=0)
    css = jnp.sum(y * y, axis=0)
    qt = ymax_ref.shape[0]
    d2 = w2_ref.shape[1]
    y3 = y.reshape(qt, K_NN, d2)
    ymax_ref[...] = jnp.max(y3, axis=1)
    ymin_ref[...] = jnp.min(y3, axis=1)

    @pl.when(i == 0)
    def _():
        s_ref[...] = jnp.zeros_like(s_ref)
        ss_ref[...] = jnp.zeros_like(ss_ref)

    s_ref[...] += jnp.broadcast_to(cs[None, :], s_ref.shape)
    ss_ref[...] += jnp.broadcast_to(css[None, :], ss_ref.shape)


def _mlp2(y1, ps, pss, g, bt, w2t, rows, qtile):
    d2 = w2t.shape[1]
    d1 = y1.shape[1]
    nq = rows // K_NN
    return pl.pallas_call(
        functools.partial(_mlp2_body, count=float(rows)),
        grid=(rows // (qtile * K_NN),),
        in_specs=[
            pl.BlockSpec((qtile * K_NN, d1), lambda i: (i, 0)),
            pl.BlockSpec((8, d1), lambda i: (0, 0)),
            pl.BlockSpec((8, d1), lambda i: (0, 0)),
            pl.BlockSpec((1, d1), lambda i: (0, 0)),
            pl.BlockSpec((1, d1), lambda i: (0, 0)),
            pl.BlockSpec(w2t.shape, lambda i: (0, 0)),
        ],
        out_specs=[
            pl.BlockSpec((qtile, d2), lambda i: (i, 0)),
            pl.BlockSpec((qtile, d2), lambda i: (i, 0)),
            pl.BlockSpec((8, d2), lambda i: (0, 0)),
            pl.BlockSpec((8, d2), lambda i: (0, 0)),
        ],
        out_shape=[
            jax.ShapeDtypeStruct((nq, d2), jnp.float32),
            jax.ShapeDtypeStruct((nq, d2), jnp.float32),
            jax.ShapeDtypeStruct((8, d2), jnp.float32),
            jax.ShapeDtypeStruct((8, d2), jnp.float32),
        ],
    )(y1, ps, pss, g, bt, w2t)


def _fin_body(ymax_ref, ymin_ref, ps_ref, pss_ref, g_ref, bt_ref, out_ref, *,
              count):
    mean = ps_ref[0:1, :] / count
    var = pss_ref[0:1, :] / count - mean * mean
    s2 = g_ref[...] * lax.rsqrt(var + 1e-5)
    t2 = bt_ref[...] - mean * s2
    o = jnp.where(s2 >= 0.0, ymax_ref[...] * s2 + t2, ymin_ref[...] * s2 + t2)
    o = jax.nn.relu(o)
    out_ref[...] = o.T[None]


def _finish(ymax, ymin, ps, pss, g, bt, b, m, d2, count):
    return pl.pallas_call(
        functools.partial(_fin_body, count=count),
        grid=(b,),
        in_specs=[
            pl.BlockSpec((m, d2), lambda i: (i, 0)),
            pl.BlockSpec((m, d2), lambda i: (i, 0)),
            pl.BlockSpec((8, d2), lambda i: (0, 0)),
            pl.BlockSpec((8, d2), lambda i: (0, 0)),
            pl.BlockSpec((1, d2), lambda i: (0, 0)),
            pl.BlockSpec((1, d2), lambda i: (0, 0)),
        ],
        out_specs=pl.BlockSpec((1, d2, m), lambda i: (i, 0, 0)),
        out_shape=jax.ShapeDtypeStruct((b, d2, m), jnp.float32),
    )(ymax, ymin, ps, pss, g, bt)


def kernel(new_xyz, xyz, feat, pre_W1, pre_b1, pre_g1, pre_bt1, pre_W2,
           pre_b2, W1, g1, bt1, W2, g2, bt2):
    B, M, _ = new_xyz.shape
    N = xyz.shape[1]
    Cin = feat.shape[1]
    D1 = W1.shape[0]
    D2 = W2.shape[0]

    ft = feat.transpose(0, 2, 1).reshape(B * N, Cin)
    xyzt = xyz.transpose(0, 2, 1)
    q = new_xyz.reshape(B * M, 3)

    s, ss = _pre_stats(ft, pre_W1.T, B * N, 1024)
    fprime = _pre_apply(ft, pre_W1.T, s, ss, pre_g1[None, :],
                        pre_bt1[None, :], pre_W2.T, pre_b2[None, :],
                        B * N, 1024)

    d = _distances(q, xyzt, B, M, N, 256)
    drows = d.reshape(B * M * (N // 128), 128)
    bmt = _blockmins(xyz, new_xyz.transpose(0, 2, 1), B, M, N, 256)
    bm = bmt.transpose(0, 2, 1).reshape(B * M, N // BLK)

    g_rows, p_rows = _sc_topk_gather(bm, drows, fprime, xyzt, B, M, N)

    w1pt = W1[:, 6:134].T
    wp = jnp.zeros((16, D1), jnp.float32)
    wp = wp.at[0:3, :].set(W1[:, 0:3].T)
    w1q = -W1[:, 0:3].T + W1[:, 3:6].T

    y1, s1sum, s1ss = _mlp1(g_rows, p_rows, q, w1pt, wp, w1q,
                            B * M * K_NN, 128)
    ymax, ymin, s2sum, s2ss = _mlp2(y1, s1sum, s1ss, g1[None, :],
                                    bt1[None, :], W2.T, B * M * K_NN, 128)
    return _finish(ymax, ymin, s2sum, s2ss, g2[None, :], bt2[None, :],
                   B, M, D2, float(B * M * K_NN))


# speedup vs baseline: 1.9866x; 1.0399x over previous
import functools

import jax
import jax.numpy as jnp
from jax import lax
from jax.experimental import pallas as pl
from jax.experimental.pallas import tpu as pltpu
from jax.experimental.pallas import tpu_sc as plsc

K_NN = 16
BLK = 64


def _a1_body(ft_ref, w_ref, s_ref, ss_ref):
    i = pl.program_id(0)
    h = jnp.dot(ft_ref[...], w_ref[...], preferred_element_type=jnp.float32)
    cs = jnp.sum(h, axis=0)
    css = jnp.sum(h * h, axis=0)

    @pl.when(i == 0)
    def _():
        s_ref[...] = jnp.zeros_like(s_ref)
        ss_ref[...] = jnp.zeros_like(ss_ref)

    s_ref[...] += jnp.broadcast_to(cs[None, :], s_ref.shape)
    ss_ref[...] += jnp.broadcast_to(css[None, :], ss_ref.shape)


def _pre_stats(ft, w1t, n_rows, tile):
    c1 = w1t.shape[1]
    s, ss = pl.pallas_call(
        _a1_body,
        grid=(n_rows // tile,),
        in_specs=[
            pl.BlockSpec((tile, ft.shape[1]), lambda i: (i, 0)),
            pl.BlockSpec(w1t.shape, lambda i: (0, 0)),
        ],
        out_specs=[
            pl.BlockSpec((8, c1), lambda i: (0, 0)),
            pl.BlockSpec((8, c1), lambda i: (0, 0)),
        ],
        out_shape=[
            jax.ShapeDtypeStruct((8, c1), jnp.float32),
            jax.ShapeDtypeStruct((8, c1), jnp.float32),
        ],
    )(ft, w1t)
    return s, ss


def _a2_body(ft_ref, w1t_ref, s_ref, ss_ref, g_ref, bt_ref, w2_ref, b2_ref,
             out_ref, *, count):
    mean = s_ref[0:1, :] / count
    var = ss_ref[0:1, :] / count - mean * mean
    scale = g_ref[...] * lax.rsqrt(var + 1e-5)
    shift = bt_ref[...] - mean * scale
    w1 = w1t_ref[...] * scale
    h = jnp.dot(ft_ref[...], w1, preferred_element_type=jnp.float32)
    h = jax.nn.relu(h + shift)
    f = jnp.dot(h, w2_ref[...], preferred_element_type=jnp.float32)
    out_ref[...] = f + b2_ref[...]


def _pre_apply(ft, w1t, s, ss, g, bt, w2t, b2, n_rows, tile):
    c1 = w1t.shape[1]
    c2 = w2t.shape[1]
    return pl.pallas_call(
        functools.partial(_a2_body, count=float(n_rows)),
        grid=(n_rows // tile,),
        in_specs=[
            pl.BlockSpec((tile, ft.shape[1]), lambda i: (i, 0)),
            pl.BlockSpec(w1t.shape, lambda i: (0, 0)),
            pl.BlockSpec((8, c1), lambda i: (0, 0)),
            pl.BlockSpec((8, c1), lambda i: (0, 0)),
            pl.BlockSpec((1, c1), lambda i: (0, 0)),
            pl.BlockSpec((1, c1), lambda i: (0, 0)),
            pl.BlockSpec(w2t.shape, lambda i: (0, 0)),
            pl.BlockSpec((1, c2), lambda i: (0, 0)),
        ],
        out_specs=pl.BlockSpec((tile, c2), lambda i: (i, 0)),
        out_shape=jax.ShapeDtypeStruct((n_rows, c2), jnp.float32),
    )(ft, w1t, s, ss, g, bt, w2t, b2)


def _dist_body(q_ref, p_ref, d_ref):
    q = q_ref[...]
    p = p_ref[0]
    qn = jnp.sum(q * q, axis=1, keepdims=True)
    pn = jnp.sum(p * p, axis=0, keepdims=True)
    d_ref[...] = qn + pn - 2.0 * jnp.dot(
        q, p, preferred_element_type=jnp.float32)


def _distances(q, xyzt, b, m, n, tile):
    steps_per_b = m // tile
    return pl.pallas_call(
        _dist_body,
        grid=(b * steps_per_b,),
        in_specs=[
            pl.BlockSpec((tile, 3), lambda i: (i, 0)),
            pl.BlockSpec((1, 3, n), lambda i, s=steps_per_b: (i // s, 0, 0)),
        ],
        out_specs=pl.BlockSpec((tile, n), lambda i: (i, 0)),
        out_shape=jax.ShapeDtypeStruct((b * m, n), jnp.float32),
    )(q, xyzt)


def _bmt_body(p_ref, qt_ref, bm_ref, *, n):
    p = p_ref[0]
    qt = qt_ref[0]
    pn = jnp.sum(p * p, axis=1, keepdims=True)
    e = pn - 2.0 * jnp.dot(p, qt, preferred_element_type=jnp.float32)
    mt = e.shape[1]
    bm = jnp.min(e.reshape(n // BLK, BLK, mt), axis=1)
    qn = jnp.sum(qt * qt, axis=0, keepdims=True)
    bm_ref[...] = bm.T + qn.T


def _blockmins(xyz, qt, b, m, n, mt):
    steps_per_b = m // mt
    return pl.pallas_call(
        functools.partial(_bmt_body, n=n),
        grid=(b * steps_per_b,),
        in_specs=[
            pl.BlockSpec((1, n, 3), lambda i, s=steps_per_b: (i // s, 0, 0)),
            pl.BlockSpec((1, 3, mt),
                         lambda i, s=steps_per_b: (i // s, 0, i % s)),
        ],
        out_specs=pl.BlockSpec((mt, n // BLK), lambda i: (i, 0)),
        out_shape=jax.ShapeDtypeStruct((b * m, n // BLK), jnp.float32),
    )(xyz, qt)


def _sc_topk_gather(bm, drows, fprime, xyzt, b, m, n):
    rows = b * m
    nblk = n // BLK
    nw = 32
    rpw = rows // nw
    mesh = plsc.VectorSubcoreMesh(
        core_axis_name="c", subcore_axis_name="s", num_cores=2,
        num_subcores=16)

    grp = 4
    ngrp = rpw // grp

    def body(bm_hbm, drows_hbm, fp_hbm, xyzt_hbm, g_hbm, p_hbm, xyz_v,
             bmg0, bmg1, blk0, blk1, gbig0, gbig1, pbig0, pbig1,
             aia0, aia1, bidx0, bidx1, fidx0, fidx1,
             semb0, semb1, semf0, semf1):
        wid = lax.axis_index("s") * 2 + lax.axis_index("c")
        base = wid * rpw
        bw = base // m
        pltpu.sync_copy(xyzt_hbm.at[bw], xyz_v)
        iota = lax.iota(jnp.int32, 16)
        zeros16 = jnp.zeros((16,), jnp.float32)
        for row in range(grp * 16):
            pbig0[row, :] = zeros16
            pbig1[row, :] = zeros16
        inf16 = jnp.full((16,), jnp.inf, jnp.float32)

        def merge(ad, ai, kd, ki):
            sd, si = plsc.sort_key_val(kd, ki)
            rd = lax.rev(sd, (0,))
            ri = lax.rev(si, (0,))
            take = ad <= rd
            md = jnp.where(take, ad, rd)
            mi = jnp.where(take, ai, ri)
            fd, fi = plsc.sort_key_val(md, mi)
            return fd, fi

        def phase_a(g, bmg, aia, bidx, blk, semb):
            rowb = base + g * grp
            pltpu.sync_copy(bm_hbm.at[pl.ds(rowb, grp)], bmg)

            def arow(j, carry):
                ad, ai = plsc.sort_key_val(bmg[j, pl.ds(0, 16)], iota)
                for c in range(1, nblk // 16):
                    kd = bmg[j, pl.ds(c * 16, 16)]
                    ad, ai = merge(ad, ai, kd, iota + (c * 16))
                aia[pl.ds(j * 16, 16)] = ai
                bidx[pl.ds(j * 16, 16)] = (
                    lax.shift_right_logical(ai, 1) + (rowb + j) * (n // 128))
                return carry

            lax.fori_loop(0, grp, arow, 0)
            pltpu.async_copy(drows_hbm.at[bidx], blk, semb)

        def phase_b(g, aia, blk, pbig, fidx, gbig, semf):

            def brow(j, carry):
                ai = aia[pl.ds(j * 16, 16)]
                ed = inf16
                ei = jnp.zeros((16,), jnp.int32)
                nbase_all = ai * BLK
                off_all = (ai & 1) * BLK
                for jj in range(16):
                    nbase = nbase_all[jj]
                    off = off_all[jj]
                    for c4 in range(BLK // 16):
                        kd = blk[j * 16 + jj, pl.ds(off + c4 * 16, 16)]
                        ed, ei = merge(ed, ei, kd, iota + (nbase + c4 * 16))
                fidx[pl.ds(j * 16, 16)] = ei + bw * n
                for c in range(3):
                    pc = plsc.load_gather(
                        xyz_v, [jnp.full((16,), c, jnp.int32), ei])
                    plsc.store_scatter(
                        pbig, [iota + j * 16,
                               jnp.full((16,), c, jnp.int32)], pc)
                return carry

            lax.fori_loop(0, grp, brow, 0)
            return pltpu.async_copy(fp_hbm.at[fidx], gbig, semf)

        def out_copies(g, gbig, pbig):
            rowb = base + g * grp
            pltpu.sync_copy(gbig, g_hbm.at[pl.ds(rowb * K_NN, grp * K_NN)])
            pltpu.sync_copy(pbig, p_hbm.at[pl.ds(rowb * K_NN, grp * K_NN)])

        def drain_b(blk, semb):
            pltpu.make_async_copy(
                drows_hbm.at[pl.ds(0, grp * 16)], blk, semb).wait()

        phase_a(0, bmg0, aia0, bidx0, blk0, semb0)

        def step(t, carry):
            a = 2 * t
            b = 2 * t + 1
            nxt = jnp.minimum(a + 2, ngrp - 1)
            phase_a(b, bmg1, aia1, bidx1, blk1, semb1)
            drain_b(blk0, semb0)
            ha = phase_b(a, aia0, blk0, pbig0, fidx0, gbig0, semf0)
            phase_a(nxt, bmg0, aia0, bidx0, blk0, semb0)
            drain_b(blk1, semb1)
            hb = phase_b(b, aia1, blk1, pbig1, fidx1, gbig1, semf1)
            ha.wait()
            out_copies(a, gbig0, pbig0)
            hb.wait()
            out_copies(b, gbig1, pbig1)
            return carry

        lax.fori_loop(0, ngrp // 2, step, 0)
        drain_b(blk0, semb0)

    fn = pl.kernel(
        body,
        out_type=(
            jax.ShapeDtypeStruct((rows * K_NN, 128), jnp.float32),
            jax.ShapeDtypeStruct((rows * K_NN, 16), jnp.float32),
        ),
        mesh=mesh,
        compiler_params=pltpu.CompilerParams(needs_layout_passes=False),
        scratch_types=[
            pltpu.VMEM((3, n), jnp.float32),
            pltpu.VMEM((grp, nblk), jnp.float32),
            pltpu.VMEM((grp, nblk), jnp.float32),
            pltpu.VMEM((grp * 16, 128), jnp.float32),
            pltpu.VMEM((grp * 16, 128), jnp.float32),
            pltpu.VMEM((grp * 16, 128), jnp.float32),
            pltpu.VMEM((grp * 16, 128), jnp.float32),
            pltpu.VMEM((grp * 16, 16), jnp.float32),
            pltpu.VMEM((grp * 16, 16), jnp.float32),
            pltpu.VMEM((grp * 16,), jnp.int32),
            pltpu.VMEM((grp * 16,), jnp.int32),
            pltpu.VMEM((grp * 16,), jnp.int32),
            pltpu.VMEM((grp * 16,), jnp.int32),
            pltpu.VMEM((grp * 16,), jnp.int32),
            pltpu.VMEM((grp * 16,), jnp.int32),
            pltpu.SemaphoreType.DMA,
            pltpu.SemaphoreType.DMA,
            pltpu.SemaphoreType.DMA,
            pltpu.SemaphoreType.DMA,
        ],
    )
    return fn(bm, drows, fprime, xyzt)


def _mlp1_body(g_ref, p_ref, q_ref, w1_ref, wp_ref, w1q_ref, y_ref, s_ref,
               ss_ref):
    i = pl.program_id(0)
    y = jnp.dot(g_ref[...], w1_ref[...], preferred_element_type=jnp.float32)
    y = y + jnp.dot(p_ref[...], wp_ref[...], preferred_element_type=jnp.float32)
    yq = jnp.dot(q_ref[...], w1q_ref[...], preferred_element_type=jnp.float32)
    qt = q_ref.shape[0]
    d1 = w1_ref.shape[1]
    y = (y.reshape(qt, K_NN, d1) + yq[:, None, :]).reshape(qt * K_NN, d1)
    y_ref[...] = y
    cs = jnp.sum(y, axis=0)
    css = jnp.sum(y * y, axis=0)

    @pl.when(i == 0)
    def _():
        s_ref[...] = jnp.zeros_like(s_ref)
        ss_ref[...] = jnp.zeros_like(ss_ref)

    s_ref[...] += jnp.broadcast_to(cs[None, :], s_ref.shape)
    ss_ref[...] += jnp.broadcast_to(css[None, :], ss_ref.shape)


def _mlp1(g2d, p2d, q, w1pt, wp, w1q, rows, qtile):
    d1 = w1pt.shape[1]
    return pl.pallas_call(
        _mlp1_body,
        grid=(rows // (qtile * K_NN),),
        in_specs=[
            pl.BlockSpec((qtile * K_NN, g2d.shape[1]), lambda i: (i, 0)),
            pl.BlockSpec((qtile * K_NN, p2d.shape[1]), lambda i: (i, 0)),
            pl.BlockSpec((qtile, 3), lambda i: (i, 0)),
            pl.BlockSpec(w1pt.shape, lambda i: (0, 0)),
            pl.BlockSpec(wp.shape, lambda i: (0, 0)),
            pl.BlockSpec(w1q.shape, lambda i: (0, 0)),
        ],
        out_specs=[
            pl.BlockSpec((qtile * K_NN, d1), lambda i: (i, 0)),
            pl.BlockSpec((8, d1), lambda i: (0, 0)),
            pl.BlockSpec((8, d1), lambda i: (0, 0)),
        ],
        out_shape=[
            jax.ShapeDtypeStruct((rows, d1), jnp.float32),
            jax.ShapeDtypeStruct((8, d1), jnp.float32),
            jax.ShapeDtypeStruct((8, d1), jnp.float32),
        ],
    )(g2d, p2d, q, w1pt, wp, w1q)


def _mlp2_body(y1_ref, ps_ref, pss_ref, g_ref, bt_ref, w2_ref, ymax_ref,
               ymin_ref, s_ref, ss_ref, *, count):
    i = pl.program_id(0)
    mean = ps_ref[0:1, :] / count
    var = pss_ref[0:1, :] / count - mean * mean
    scale = g_ref[...] * lax.rsqrt(var + 1e-5)
    shift = bt_ref[...] - mean * scale
    z = jax.nn.relu(y1_ref[...] * scale + shift)
    y = jnp.dot(z, w2_ref[...], preferred_element_type=jnp.float32)
    cs = jnp.sum(y, axis=0)
    css = jnp.sum(y * y, axis=0)
    qt = ymax_ref.shape[0]
    d2 = w2_ref.shape[1]
    y3 = y.reshape(qt, K_NN, d2)
    ymax_ref[...] = jnp.max(y3, axis=1)
    ymin_ref[...] = jnp.min(y3, axis=1)

    @pl.when(i == 0)
    def _():
        s_ref[...] = jnp.zeros_like(s_ref)
        ss_ref[...] = jnp.zeros_like(ss_ref)

    s_ref[...] += jnp.broadcast_to(cs[None, :], s_ref.shape)
    ss_ref[...] += jnp.broadcast_to(css[None, :], ss_ref.shape)


def _mlp2(y1, ps, pss, g, bt, w2t, rows, qtile):
    d2 = w2t.shape[1]
    d1 = y1.shape[1]
    nq = rows // K_NN
    return pl.pallas_call(
        functools.partial(_mlp2_body, count=float(rows)),
        grid=(rows // (qtile * K_NN),),
        in_specs=[
            pl.BlockSpec((qtile * K_NN, d1), lambda i: (i, 0)),
            pl.BlockSpec((8, d1), lambda i: (0, 0)),
            pl.BlockSpec((8, d1), lambda i: (0, 0)),
            pl.BlockSpec((1, d1), lambda i: (0, 0)),
            pl.BlockSpec((1, d1), lambda i: (0, 0)),
            pl.BlockSpec(w2t.shape, lambda i: (0, 0)),
        ],
        out_specs=[
            pl.BlockSpec((qtile, d2), lambda i: (i, 0)),
            pl.BlockSpec((qtile, d2), lambda i: (i, 0)),
            pl.BlockSpec((8, d2), lambda i: (0, 0)),
            pl.BlockSpec((8, d2), lambda i: (0, 0)),
        ],
        out_shape=[
            jax.ShapeDtypeStruct((nq, d2), jnp.float32),
            jax.ShapeDtypeStruct((nq, d2), jnp.float32),
            jax.ShapeDtypeStruct((8, d2), jnp.float32),
            jax.ShapeDtypeStruct((8, d2), jnp.float32),
        ],
    )(y1, ps, pss, g, bt, w2t)


def _fin_body(ymax_ref, ymin_ref, ps_ref, pss_ref, g_ref, bt_ref, out_ref, *,
              count):
    mean = ps_ref[0:1, :] / count
    var = pss_ref[0:1, :] / count - mean * mean
    s2 = g_ref[...] * lax.rsqrt(var + 1e-5)
    t2 = bt_ref[...] - mean * s2
    o = jnp.where(s2 >= 0.0, ymax_ref[...] * s2 + t2, ymin_ref[...] * s2 + t2)
    o = jax.nn.relu(o)
    out_ref[...] = o.T[None]


def _finish(ymax, ymin, ps, pss, g, bt, b, m, d2, count):
    return pl.pallas_call(
        functools.partial(_fin_body, count=count),
        grid=(b,),
        in_specs=[
            pl.BlockSpec((m, d2), lambda i: (i, 0)),
            pl.BlockSpec((m, d2), lambda i: (i, 0)),
            pl.BlockSpec((8, d2), lambda i: (0, 0)),
            pl.BlockSpec((8, d2), lambda i: (0, 0)),
            pl.BlockSpec((1, d2), lambda i: (0, 0)),
            pl.BlockSpec((1, d2), lambda i: (0, 0)),
        ],
        out_specs=pl.BlockSpec((1, d2, m), lambda i: (i, 0, 0)),
        out_shape=jax.ShapeDtypeStruct((b, d2, m), jnp.float32),
    )(ymax, ymin, ps, pss, g, bt)


def kernel(new_xyz, xyz, feat, pre_W1, pre_b1, pre_g1, pre_bt1, pre_W2,
           pre_b2, W1, g1, bt1, W2, g2, bt2):
    B, M, _ = new_xyz.shape
    N = xyz.shape[1]
    Cin = feat.shape[1]
    D1 = W1.shape[0]
    D2 = W2.shape[0]

    ft = feat.transpose(0, 2, 1).reshape(B * N, Cin)
    xyzt = xyz.transpose(0, 2, 1)
    q = new_xyz.reshape(B * M, 3)

    s, ss = _pre_stats(ft, pre_W1.T, B * N, 1024)
    fprime = _pre_apply(ft, pre_W1.T, s, ss, pre_g1[None, :],
                        pre_bt1[None, :], pre_W2.T, pre_b2[None, :],
                        B * N, 1024)

    d = _distances(q, xyzt, B, M, N, min(512, M))
    drows = d.reshape(B * M * (N // 128), 128)
    bm = _blockmins(xyz, new_xyz.transpose(0, 2, 1), B, M, N, 256)

    g_rows, p_rows = _sc_topk_gather(bm, drows, fprime, xyzt, B, M, N)

    w1pt = W1[:, 6:134].T
    wp = jnp.zeros((16, D1), jnp.float32)
    wp = wp.at[0:3, :].set(W1[:, 0:3].T)
    w1q = -W1[:, 0:3].T + W1[:, 3:6].T

    y1, s1sum, s1ss = _mlp1(g_rows, p_rows, q, w1pt, wp, w1q,
                            B * M * K_NN, 256)
    ymax, ymin, s2sum, s2ss = _mlp2(y1, s1sum, s1ss, g1[None, :],
                                    bt1[None, :], W2.T, B * M * K_NN, 256)
    return _finish(ymax, ymin, s2sum, s2ss, g2[None, :], bt2[None, :],
                   B, M, D2, float(B * M * K_NN))


# speedup vs baseline: 2.0253x; 1.0195x over previous
import functools

import jax
import jax.numpy as jnp
from jax import lax
from jax.experimental import pallas as pl
from jax.experimental.pallas import tpu as pltpu
from jax.experimental.pallas import tpu_sc as plsc

K_NN = 16
BLK = 64


def _a1_body(ft_ref, w_ref, s_ref, ss_ref):
    i = pl.program_id(0)
    h = jnp.dot(ft_ref[...], w_ref[...], preferred_element_type=jnp.float32)
    cs = jnp.sum(h, axis=0)
    css = jnp.sum(h * h, axis=0)

    @pl.when(i == 0)
    def _():
        s_ref[...] = jnp.zeros_like(s_ref)
        ss_ref[...] = jnp.zeros_like(ss_ref)

    s_ref[...] += jnp.broadcast_to(cs[None, :], s_ref.shape)
    ss_ref[...] += jnp.broadcast_to(css[None, :], ss_ref.shape)


def _pre_stats(ft, w1t, n_rows, tile):
    c1 = w1t.shape[1]
    s, ss = pl.pallas_call(
        _a1_body,
        grid=(n_rows // tile,),
        in_specs=[
            pl.BlockSpec((tile, ft.shape[1]), lambda i: (i, 0)),
            pl.BlockSpec(w1t.shape, lambda i: (0, 0)),
        ],
        out_specs=[
            pl.BlockSpec((8, c1), lambda i: (0, 0)),
            pl.BlockSpec((8, c1), lambda i: (0, 0)),
        ],
        out_shape=[
            jax.ShapeDtypeStruct((8, c1), jnp.float32),
            jax.ShapeDtypeStruct((8, c1), jnp.float32),
        ],
    )(ft, w1t)
    return s, ss


def _a2_body(ft_ref, w1t_ref, s_ref, ss_ref, g_ref, bt_ref, w2_ref, b2_ref,
             out_ref, *, count):
    mean = s_ref[0:1, :] / count
    var = ss_ref[0:1, :] / count - mean * mean
    scale = g_ref[...] * lax.rsqrt(var + 1e-5)
    shift = bt_ref[...] - mean * scale
    w1 = w1t_ref[...] * scale
    h = jnp.dot(ft_ref[...], w1, preferred_element_type=jnp.float32)
    h = jax.nn.relu(h + shift)
    f = jnp.dot(h, w2_ref[...], preferred_element_type=jnp.float32)
    out_ref[...] = f + b2_ref[...]


def _pre_apply(ft, w1t, s, ss, g, bt, w2t, b2, n_rows, tile):
    c1 = w1t.shape[1]
    c2 = w2t.shape[1]
    return pl.pallas_call(
        functools.partial(_a2_body, count=float(n_rows)),
        grid=(n_rows // tile,),
        in_specs=[
            pl.BlockSpec((tile, ft.shape[1]), lambda i: (i, 0)),
            pl.BlockSpec(w1t.shape, lambda i: (0, 0)),
            pl.BlockSpec((8, c1), lambda i: (0, 0)),
            pl.BlockSpec((8, c1), lambda i: (0, 0)),
            pl.BlockSpec((1, c1), lambda i: (0, 0)),
            pl.BlockSpec((1, c1), lambda i: (0, 0)),
            pl.BlockSpec(w2t.shape, lambda i: (0, 0)),
            pl.BlockSpec((1, c2), lambda i: (0, 0)),
        ],
        out_specs=pl.BlockSpec((tile, c2), lambda i: (i, 0)),
        out_shape=jax.ShapeDtypeStruct((n_rows, c2), jnp.float32),
    )(ft, w1t, s, ss, g, bt, w2t, b2)


def _dist_body(q_ref, p_ref, d_ref):
    q = q_ref[...]
    p = p_ref[0]
    qn = jnp.sum(q * q, axis=1, keepdims=True)
    pn = jnp.sum(p * p, axis=0, keepdims=True)
    d_ref[...] = qn + pn - 2.0 * jnp.dot(
        q, p, preferred_element_type=jnp.float32)


def _distances(q, xyzt, b, m, n, tile):
    steps_per_b = m // tile
    return pl.pallas_call(
        _dist_body,
        grid=(b * steps_per_b,),
        in_specs=[
            pl.BlockSpec((tile, 3), lambda i: (i, 0)),
            pl.BlockSpec((1, 3, n), lambda i, s=steps_per_b: (i // s, 0, 0)),
        ],
        out_specs=pl.BlockSpec((tile, n), lambda i: (i, 0)),
        out_shape=jax.ShapeDtypeStruct((b * m, n), jnp.float32),
    )(q, xyzt)


def _bmt_body(p_ref, qt_ref, bm_ref, *, n):
    p = p_ref[0]
    qt = qt_ref[0]
    pn = jnp.sum(p * p, axis=1, keepdims=True)
    e = pn - 2.0 * jnp.dot(p, qt, preferred_element_type=jnp.float32)
    mt = e.shape[1]
    bm = jnp.min(e.reshape(n // BLK, BLK, mt), axis=1)
    qn = jnp.sum(qt * qt, axis=0, keepdims=True)
    bm_ref[...] = bm.T + qn.T


def _blockmins(xyz, qt, b, m, n, mt):
    steps_per_b = m // mt
    return pl.pallas_call(
        functools.partial(_bmt_body, n=n),
        grid=(b * steps_per_b,),
        in_specs=[
            pl.BlockSpec((1, n, 3), lambda i, s=steps_per_b: (i // s, 0, 0)),
            pl.BlockSpec((1, 3, mt),
                         lambda i, s=steps_per_b: (i // s, 0, i % s)),
        ],
        out_specs=pl.BlockSpec((mt, n // BLK), lambda i: (i, 0)),
        out_shape=jax.ShapeDtypeStruct((b * m, n // BLK), jnp.float32),
    )(xyz, qt)


def _sc_topk_gather(bm, drows, fprime, xyzt, b, m, n):
    rows = b * m
    nblk = n // BLK
    nw = 32
    rpw = rows // nw
    mesh = plsc.VectorSubcoreMesh(
        core_axis_name="c", subcore_axis_name="s", num_cores=2,
        num_subcores=16)

    grp = 4
    ngrp = rpw // grp

    def body(bm_hbm, drows_hbm, fp_hbm, xyzt_hbm, g_hbm, p_hbm, xyz_v,
             bmg0, bmg1, blk0, blk1, gbig0, gbig1, pbig0, pbig1,
             aia0, aia1, bidx0, bidx1, fidx0, fidx1,
             semb0, semb1, semf0, semf1, semg0, semg1, semp0, semp1):
        wid = lax.axis_index("s") * 2 + lax.axis_index("c")
        base = wid * rpw
        bw = base // m
        pltpu.sync_copy(xyzt_hbm.at[bw], xyz_v)
        iota = lax.iota(jnp.int32, 16)
        zeros16 = jnp.zeros((16,), jnp.float32)
        for row in range(grp * 16):
            pbig0[row, :] = zeros16
            pbig1[row, :] = zeros16
        inf16 = jnp.full((16,), jnp.inf, jnp.float32)

        def merge(ad, ai, kd, ki):
            sd, si = plsc.sort_key_val(kd, ki)
            rd = lax.rev(sd, (0,))
            ri = lax.rev(si, (0,))
            take = ad <= rd
            md = jnp.where(take, ad, rd)
            mi = jnp.where(take, ai, ri)
            fd, fi = plsc.sort_key_val(md, mi)
            return fd, fi

        def phase_a(g, bmg, aia, bidx, blk, semb):
            rowb = base + g * grp
            pltpu.sync_copy(bm_hbm.at[pl.ds(rowb, grp)], bmg)

            def arow(j, carry):
                ad, ai = plsc.sort_key_val(bmg[j, pl.ds(0, 16)], iota)
                for c in range(1, nblk // 16):
                    kd = bmg[j, pl.ds(c * 16, 16)]
                    ad, ai = merge(ad, ai, kd, iota + (c * 16))
                aia[pl.ds(j * 16, 16)] = ai
                bidx[pl.ds(j * 16, 16)] = (
                    lax.shift_right_logical(ai, 1) + (rowb + j) * (n // 128))
                return carry

            lax.fori_loop(0, grp, arow, 0)
            pltpu.async_copy(drows_hbm.at[bidx], blk, semb)

        def phase_b(g, aia, blk, pbig, fidx, gbig, semf):

            def brow(j, carry):
                ai = aia[pl.ds(j * 16, 16)]
                ed = inf16
                ei = jnp.zeros((16,), jnp.int32)
                nbase_all = ai * BLK
                off_all = (ai & 1) * BLK
                for jj in range(16):
                    nbase = nbase_all[jj]
                    off = off_all[jj]
                    for c4 in range(BLK // 16):
                        kd = blk[j * 16 + jj, pl.ds(off + c4 * 16, 16)]
                        ed, ei = merge(ed, ei, kd, iota + (nbase + c4 * 16))
                fidx[pl.ds(j * 16, 16)] = ei + bw * n
                for c in range(3):
                    pc = plsc.load_gather(
                        xyz_v, [jnp.full((16,), c, jnp.int32), ei])
                    plsc.store_scatter(
                        pbig, [iota + j * 16,
                               jnp.full((16,), c, jnp.int32)], pc)
                return carry

            lax.fori_loop(0, grp, brow, 0)
            return pltpu.async_copy(fp_hbm.at[fidx], gbig, semf)

        def out_copies(g, gbig, pbig, semg, semp):
            rowb = base + g * grp
            pltpu.async_copy(
                gbig, g_hbm.at[pl.ds(rowb * K_NN, grp * K_NN)], semg)
            pltpu.async_copy(
                pbig, p_hbm.at[pl.ds(rowb * K_NN, grp * K_NN)], semp)

        def drain_b(blk, semb):
            pltpu.make_async_copy(
                drows_hbm.at[pl.ds(0, grp * 16)], blk, semb).wait()

        def drain_out(gbig, pbig, semg, semp):
            pltpu.make_async_copy(
                gbig, g_hbm.at[pl.ds(0, grp * K_NN)], semg).wait()
            pltpu.make_async_copy(
                pbig, p_hbm.at[pl.ds(0, grp * K_NN)], semp).wait()

        phase_a(0, bmg0, aia0, bidx0, blk0, semb0)

        def step(t, carry):
            a = 2 * t
            b = 2 * t + 1
            nxt = jnp.minimum(a + 2, ngrp - 1)
            phase_a(b, bmg1, aia1, bidx1, blk1, semb1)
            drain_b(blk0, semb0)

            @pl.when(t > 0)
            def _():
                drain_out(gbig0, pbig0, semg0, semp0)

            ha = phase_b(a, aia0, blk0, pbig0, fidx0, gbig0, semf0)
            phase_a(nxt, bmg0, aia0, bidx0, blk0, semb0)
            drain_b(blk1, semb1)

            @pl.when(t > 0)
            def _():
                drain_out(gbig1, pbig1, semg1, semp1)

            hb = phase_b(b, aia1, blk1, pbig1, fidx1, gbig1, semf1)
            ha.wait()
            out_copies(a, gbig0, pbig0, semg0, semp0)
            hb.wait()
            out_copies(b, gbig1, pbig1, semg1, semp1)
            return carry

        lax.fori_loop(0, ngrp // 2, step, 0)
        drain_b(blk0, semb0)
        drain_out(gbig0, pbig0, semg0, semp0)
        drain_out(gbig1, pbig1, semg1, semp1)

    fn = pl.kernel(
        body,
        out_type=(
            jax.ShapeDtypeStruct((rows * K_NN, 128), jnp.float32),
            jax.ShapeDtypeStruct((rows * K_NN, 16), jnp.float32),
        ),
        mesh=mesh,
        compiler_params=pltpu.CompilerParams(needs_layout_passes=False),
        scratch_types=[
            pltpu.VMEM((3, n), jnp.float32),
            pltpu.VMEM((grp, nblk), jnp.float32),
            pltpu.VMEM((grp, nblk), jnp.float32),
            pltpu.VMEM((grp * 16, 128), jnp.float32),
            pltpu.VMEM((grp * 16, 128), jnp.float32),
            pltpu.VMEM((grp * 16, 128), jnp.float32),
            pltpu.VMEM((grp * 16, 128), jnp.float32),
            pltpu.VMEM((grp * 16, 16), jnp.float32),
            pltpu.VMEM((grp * 16, 16), jnp.float32),
            pltpu.VMEM((grp * 16,), jnp.int32),
            pltpu.VMEM((grp * 16,), jnp.int32),
            pltpu.VMEM((grp * 16,), jnp.int32),
            pltpu.VMEM((grp * 16,), jnp.int32),
            pltpu.VMEM((grp * 16,), jnp.int32),
            pltpu.VMEM((grp * 16,), jnp.int32),
            pltpu.SemaphoreType.DMA,
            pltpu.SemaphoreType.DMA,
            pltpu.SemaphoreType.DMA,
            pltpu.SemaphoreType.DMA,
            pltpu.SemaphoreType.DMA,
            pltpu.SemaphoreType.DMA,
            pltpu.SemaphoreType.DMA,
            pltpu.SemaphoreType.DMA,
        ],
    )
    return fn(bm, drows, fprime, xyzt)


def _mlp1_body(g_ref, p_ref, q_ref, w1_ref, wp_ref, w1q_ref, y_ref, s_ref,
               ss_ref):
    i = pl.program_id(0)
    y = jnp.dot(g_ref[...], w1_ref[...], preferred_element_type=jnp.float32)
    y = y + jnp.dot(p_ref[...], wp_ref[...], preferred_element_type=jnp.float32)
    yq = jnp.dot(q_ref[...], w1q_ref[...], preferred_element_type=jnp.float32)
    qt = q_ref.shape[0]
    d1 = w1_ref.shape[1]
    y = (y.reshape(qt, K_NN, d1) + yq[:, None, :]).reshape(qt * K_NN, d1)
    y_ref[...] = y
    cs = jnp.sum(y, axis=0)
    css = jnp.sum(y * y, axis=0)

    @pl.when(i == 0)
    def _():
        s_ref[...] = jnp.zeros_like(s_ref)
        ss_ref[...] = jnp.zeros_like(ss_ref)

    s_ref[...] += jnp.broadcast_to(cs[None, :], s_ref.shape)
    ss_ref[...] += jnp.broadcast_to(css[None, :], ss_ref.shape)


def _mlp1(g2d, p2d, q, w1pt, wp, w1q, rows, qtile):
    d1 = w1pt.shape[1]
    return pl.pallas_call(
        _mlp1_body,
        grid=(rows // (qtile * K_NN),),
        in_specs=[
            pl.BlockSpec((qtile * K_NN, g2d.shape[1]), lambda i: (i, 0)),
            pl.BlockSpec((qtile * K_NN, p2d.shape[1]), lambda i: (i, 0)),
            pl.BlockSpec((qtile, 3), lambda i: (i, 0)),
            pl.BlockSpec(w1pt.shape, lambda i: (0, 0)),
            pl.BlockSpec(wp.shape, lambda i: (0, 0)),
            pl.BlockSpec(w1q.shape, lambda i: (0, 0)),
        ],
        out_specs=[
            pl.BlockSpec((qtile * K_NN, d1), lambda i: (i, 0)),
            pl.BlockSpec((8, d1), lambda i: (0, 0)),
            pl.BlockSpec((8, d1), lambda i: (0, 0)),
        ],
        out_shape=[
            jax.ShapeDtypeStruct((rows, d1), jnp.float32),
            jax.ShapeDtypeStruct((8, d1), jnp.float32),
            jax.ShapeDtypeStruct((8, d1), jnp.float32),
        ],
    )(g2d, p2d, q, w1pt, wp, w1q)


def _mlp2_body(y1_ref, ps_ref, pss_ref, g_ref, bt_ref, w2_ref, ymax_ref,
               ymin_ref, s_ref, ss_ref, *, count):
    i = pl.program_id(0)
    mean = ps_ref[0:1, :] / count
    var = pss_ref[0:1, :] / count - mean * mean
    scale = g_ref[...] * lax.rsqrt(var + 1e-5)
    shift = bt_ref[...] - mean * scale
    z = jax.nn.relu(y1_ref[...] * scale + shift)
    y = jnp.dot(z, w2_ref[...], preferred_element_type=jnp.float32)
    cs = jnp.sum(y, axis=0)
    css = jnp.sum(y * y, axis=0)
    qt = ymax_ref.shape[0]
    d2 = w2_ref.shape[1]
    y3 = y.reshape(qt, K_NN, d2)
    ymax_ref[...] = jnp.max(y3, axis=1)
    ymin_ref[...] = jnp.min(y3, axis=1)

    @pl.when(i == 0)
    def _():
        s_ref[...] = jnp.zeros_like(s_ref)
        ss_ref[...] = jnp.zeros_like(ss_ref)

    s_ref[...] += jnp.broadcast_to(cs[None, :], s_ref.shape)
    ss_ref[...] += jnp.broadcast_to(css[None, :], ss_ref.shape)


def _mlp2(y1, ps, pss, g, bt, w2t, rows, qtile):
    d2 = w2t.shape[1]
    d1 = y1.shape[1]
    nq = rows // K_NN
    return pl.pallas_call(
        functools.partial(_mlp2_body, count=float(rows)),
        grid=(rows // (qtile * K_NN),),
        in_specs=[
            pl.BlockSpec((qtile * K_NN, d1), lambda i: (i, 0)),
            pl.BlockSpec((8, d1), lambda i: (0, 0)),
            pl.BlockSpec((8, d1), lambda i: (0, 0)),
            pl.BlockSpec((1, d1), lambda i: (0, 0)),
            pl.BlockSpec((1, d1), lambda i: (0, 0)),
            pl.BlockSpec(w2t.shape, lambda i: (0, 0)),
        ],
        out_specs=[
            pl.BlockSpec((qtile, d2), lambda i: (i, 0)),
            pl.BlockSpec((qtile, d2), lambda i: (i, 0)),
            pl.BlockSpec((8, d2), lambda i: (0, 0)),
            pl.BlockSpec((8, d2), lambda i: (0, 0)),
        ],
        out_shape=[
            jax.ShapeDtypeStruct((nq, d2), jnp.float32),
            jax.ShapeDtypeStruct((nq, d2), jnp.float32),
            jax.ShapeDtypeStruct((8, d2), jnp.float32),
            jax.ShapeDtypeStruct((8, d2), jnp.float32),
        ],
    )(y1, ps, pss, g, bt, w2t)


def _fin_body(ymax_ref, ymin_ref, ps_ref, pss_ref, g_ref, bt_ref, out_ref, *,
              count):
    mean = ps_ref[0:1, :] / count
    var = pss_ref[0:1, :] / count - mean * mean
    s2 = g_ref[...] * lax.rsqrt(var + 1e-5)
    t2 = bt_ref[...] - mean * s2
    o = jnp.where(s2 >= 0.0, ymax_ref[...] * s2 + t2, ymin_ref[...] * s2 + t2)
    o = jax.nn.relu(o)
    out_ref[...] = o.T[None]


def _finish(ymax, ymin, ps, pss, g, bt, b, m, d2, count):
    return pl.pallas_call(
        functools.partial(_fin_body, count=count),
        grid=(b,),
        in_specs=[
            pl.BlockSpec((m, d2), lambda i: (i, 0)),
            pl.BlockSpec((m, d2), lambda i: (i, 0)),
            pl.BlockSpec((8, d2), lambda i: (0, 0)),
            pl.BlockSpec((8, d2), lambda i: (0, 0)),
            pl.BlockSpec((1, d2), lambda i: (0, 0)),
            pl.BlockSpec((1, d2), lambda i: (0, 0)),
        ],
        out_specs=pl.BlockSpec((1, d2, m), lambda i: (i, 0, 0)),
        out_shape=jax.ShapeDtypeStruct((b, d2, m), jnp.float32),
    )(ymax, ymin, ps, pss, g, bt)


def kernel(new_xyz, xyz, feat, pre_W1, pre_b1, pre_g1, pre_bt1, pre_W2,
           pre_b2, W1, g1, bt1, W2, g2, bt2):
    B, M, _ = new_xyz.shape
    N = xyz.shape[1]
    Cin = feat.shape[1]
    D1 = W1.shape[0]
    D2 = W2.shape[0]

    ft = feat.transpose(0, 2, 1).reshape(B * N, Cin)
    xyzt = xyz.transpose(0, 2, 1)
    q = new_xyz.reshape(B * M, 3)

    s, ss = _pre_stats(ft, pre_W1.T, B * N, 1024)
    fprime = _pre_apply(ft, pre_W1.T, s, ss, pre_g1[None, :],
                        pre_bt1[None, :], pre_W2.T, pre_b2[None, :],
                        B * N, 1024)

    d = _distances(q, xyzt, B, M, N, min(512, M))
    drows = d.reshape(B * M * (N // 128), 128)
    bm = _blockmins(xyz, new_xyz.transpose(0, 2, 1), B, M, N, 256)

    g_rows, p_rows = _sc_topk_gather(bm, drows, fprime, xyzt, B, M, N)

    w1pt = W1[:, 6:134].T
    wp = jnp.zeros((16, D1), jnp.float32)
    wp = wp.at[0:3, :].set(W1[:, 0:3].T)
    w1q = -W1[:, 0:3].T + W1[:, 3:6].T

    y1, s1sum, s1ss = _mlp1(g_rows, p_rows, q, w1pt, wp, w1q,
                            B * M * K_NN, 256)
    ymax, ymin, s2sum, s2ss = _mlp2(y1, s1sum, s1ss, g1[None, :],
                                    bt1[None, :], W2.T, B * M * K_NN, 256)
    return _finish(ymax, ymin, s2sum, s2ss, g2[None, :], bt2[None, :],
                   B, M, D2, float(B * M * K_NN))
